# Initial kernel scaffold; baseline (speedup 1.0000x reference)
import jax
import jax.numpy as jnp
from jax.experimental import pallas as pl


def kernel(dist, embed_q_weight):
    raise NotImplementedError("write your pallas kernel here")


# speedup vs baseline: 60.7789x; 60.7789x over previous
import functools

import jax
import jax.numpy as jnp
from jax import lax
from jax.experimental import pallas as pl
from jax.experimental.pallas import tpu as pltpu
from jax.experimental.pallas import tpu_sc as plsc

EMBED = 64
ROWS = 65
N = 16384 * 50
NW = 32
N_TILE = N // NW
CHUNK = 512
NCHUNK = N_TILE // CHUNK


def _sc_body(dist_hbm, table_hbm, out_hbm, dist_v, table_v, out_v, sem):
    wid = lax.axis_index("s") * 2 + lax.axis_index("c")
    base = wid * N_TILE
    pltpu.sync_copy(table_hbm, table_v)
    pltpu.sync_copy(dist_hbm.at[pl.ds(base, N_TILE)], dist_v)

    def chunk_body(g, carry):
        off = g * CHUNK

        def grp_body(j, c2):
            d = dist_v[pl.ds(off + j * 16, 16)]
            f = d * 64.0
            l = f.astype(jnp.int32)
            frac = f - l.astype(jnp.float32)
            li = l * EMBED
            for k in range(16):
                b = li[k]
                fb = jnp.full((16,), frac[k], jnp.float32)
                for c in range(EMBED // 16):
                    lo = table_v[pl.ds(b + c * 16, 16)]
                    hi = table_v[pl.ds(b + EMBED + c * 16, 16)]
                    out_v[pl.ds((j * 16 + k) * EMBED + c * 16, 16)] = (
                        lo + fb * (hi - lo))
            return c2

        lax.fori_loop(0, CHUNK // 16, grp_body, 0)
        pltpu.sync_copy(out_v, out_hbm.at[pl.ds((base + off) * EMBED,
                                                CHUNK * EMBED)])
        return carry

    lax.fori_loop(0, NCHUNK, chunk_body, 0)


_sc_kernel = functools.partial(
    pl.kernel,
    out_type=jax.ShapeDtypeStruct((N * EMBED,), jnp.float32),
    mesh=plsc.VectorSubcoreMesh(core_axis_name="c", subcore_axis_name="s"),
    scratch_types=[
        pltpu.VMEM((N_TILE,), jnp.float32),
        pltpu.VMEM((ROWS * EMBED,), jnp.float32),
        pltpu.VMEM((CHUNK * EMBED,), jnp.float32),
        pltpu.SemaphoreType.DMA,
    ],
)(_sc_body)


def kernel(dist, embed_q_weight):
    d = dist.reshape(-1).astype(jnp.float32)
    t = embed_q_weight.reshape(-1)
    out = _sc_kernel(d, t)
    return out.reshape(N, EMBED)
